# Initial kernel scaffold; baseline (speedup 1.0000x reference)
import jax
import jax.numpy as jnp
from jax.experimental import pallas as pl


def kernel(points, new_points, features):
    raise NotImplementedError("write your pallas kernel here")


# speedup vs baseline: 8.2873x; 8.2873x over previous
import functools

import jax
import jax.numpy as jnp
from jax import lax
from jax.experimental import pallas as pl
from jax.experimental.pallas import tpu as pltpu
from jax.experimental.pallas import tpu_sc as plsc

_B, _N, _M, _C, _S = 4, 8192, 2048, 64, 16
_L = 16
_NW = 32
_QPW = (_B * _M) // _NW
_WPB = _NW // _B
_NV = _N // _L
_QG = 16
_CH = 3 + _C


def _bf16_round(v):
    u = plsc.bitcast(v, jnp.int32)
    r = (u + jnp.int32(0x7FFF) + ((u >> 16) & 1)) & jnp.int32(-65536)
    return plsc.bitcast(r, jnp.float32)


def _sc_body(points, newpts, featsT, out,
             x_ref, y_ref, z_ref, xb_ref, yb_ref, zb_ref, xx_ref,
             d_ref, sd_ref, si_ref,
             fbuf_ref, stage_ref, idx_ref, q_ref, sem_in, sem_f):
    c = lax.axis_index("c")
    s = lax.axis_index("s")
    wid = s * 2 + c
    b = wid // _WPB
    q0 = (wid % _WPB) * _QPW

    pltpu.sync_copy(points.at[b, 0], x_ref)
    pltpu.sync_copy(points.at[b, 1], y_ref)
    pltpu.sync_copy(points.at[b, 2], z_ref)
    pltpu.sync_copy(newpts.at[b, :, pl.ds(q0, _QPW)], q_ref)

    @pl.loop(0, _NV)
    def _pre(i):
        sl = pl.ds(i * _L, _L)
        xv = x_ref[sl]
        yv = y_ref[sl]
        zv = z_ref[sl]
        xx_ref[sl] = xv * xv + yv * yv + zv * zv
        xb_ref[sl] = _bf16_round(xv)
        yb_ref[sl] = _bf16_round(yv)
        zb_ref[sl] = _bf16_round(zv)

    iota = lax.iota(jnp.int32, _L)
    inf16 = jnp.full((_L,), jnp.inf, jnp.float32)
    zeros16i = jnp.zeros((_L,), jnp.int32)
    cbase = iota * _C

    @pl.loop(0, _QPW, step=_QG)
    def _group(g):
        @pl.loop(0, _QG)
        def _query(qi):
            mm = jnp.full((_L,), g + qi, jnp.int32)
            qxv = plsc.load_gather(q_ref, [jnp.zeros((_L,), jnp.int32), mm])
            qyv = plsc.load_gather(q_ref, [jnp.ones((_L,), jnp.int32), mm])
            qzv = plsc.load_gather(q_ref, [jnp.full((_L,), 2, jnp.int32), mm])
            qxb = _bf16_round(qxv)
            qyb = _bf16_round(qyv)
            qzb = _bf16_round(qzv)
            yyv = qxv * qxv + qyv * qyv + qzv * qzv

            def p1(i, m16):
                sl = pl.ds(i * _L, _L)
                t = xb_ref[sl] * qxb + yb_ref[sl] * qyb + zb_ref[sl] * qzb
                d = (xx_ref[sl] + yyv) - 2.0 * t
                d_ref[sl] = d
                return jnp.minimum(m16, d)

            m16 = lax.fori_loop(0, _NV, p1, inf16)
            t0 = jnp.full((_L,), jnp.max(m16))

            def p2(i, carry):
                off, iv = carry
                d = d_ref[pl.ds(i * _L, _L)]
                msk = d <= t0
                pos = off + plsc.cumsum(jnp.where(msk, 1, 0)) - 1
                plsc.store_scatter(sd_ref, [pos], d, mask=msk)
                plsc.store_scatter(si_ref, [pos], iv, mask=msk)
                return off + plsc.all_reduce_population_count(msk), iv + _L

            off, _ = lax.fori_loop(0, _NV, p2, (zeros16i, iota))
            total = jnp.max(off)
            plsc.store_scatter(sd_ref, [off + iota], inf16)
            plsc.store_scatter(si_ref, [off + iota], zeros16i)

            def p3(i, carry):
                td, ti = carry
                cd = sd_ref[pl.ds(i * _L, _L)]
                ci = si_ref[pl.ds(i * _L, _L)]
                cd, ci = plsc.sort_key_val(cd, ci)
                rd = lax.rev(cd, (0,))
                ri = lax.rev(ci, (0,))
                keep = td <= rd
                ld = jnp.minimum(td, rd)
                li = jnp.where(keep, ti, ri)
                nd, ni = plsc.sort_key_val(ld, li)
                return nd, ni

            nvregs = (total + _L - 1) >> 4
            td, ti = lax.fori_loop(0, nvregs, p3, (inf16, zeros16i))

            stage_ref[0, qi] = plsc.load_gather(x_ref, [ti]) - qxv
            stage_ref[1, qi] = plsc.load_gather(y_ref, [ti]) - qyv
            stage_ref[2, qi] = plsc.load_gather(z_ref, [ti]) - qzv
            idx_ref[...] = ti
            pltpu.async_copy(featsT.at[b].at[idx_ref], fbuf_ref, sem_f).wait()

            @pl.loop(0, _C)
            def _chan(ch):
                col = plsc.load_gather(fbuf_ref, [iota, jnp.full((_L,), ch)])
                stage_ref[3 + ch, qi] = col

        pltpu.sync_copy(
            stage_ref,
            out.at[pl.ds(b * _CH, _CH), pl.ds(q0 + g, _QG), :])


@jax.jit
def kernel(points, new_points, features):
    featsT = jnp.transpose(features, (0, 2, 1))
    mesh = plsc.VectorSubcoreMesh(core_axis_name="c", subcore_axis_name="s")
    run = pl.kernel(
        _sc_body,
        out_type=jax.ShapeDtypeStruct((_B * _CH, _M, _S), jnp.float32),
        mesh=mesh,
        compiler_params=pltpu.CompilerParams(
            use_tc_tiling_on_sc=False, needs_layout_passes=False),
        scratch_types=[
            pltpu.VMEM((_N,), jnp.float32),
            pltpu.VMEM((_N,), jnp.float32),
            pltpu.VMEM((_N,), jnp.float32),
            pltpu.VMEM((_N,), jnp.float32),
            pltpu.VMEM((_N,), jnp.float32),
            pltpu.VMEM((_N,), jnp.float32),
            pltpu.VMEM((_N,), jnp.float32),
            pltpu.VMEM((_N,), jnp.float32),
            pltpu.VMEM((_N + _L,), jnp.float32),
            pltpu.VMEM((_N + _L,), jnp.int32),
            pltpu.VMEM((_S, _C), jnp.float32),
            pltpu.VMEM((_CH, _QG, _S), jnp.float32),
            pltpu.VMEM((_S,), jnp.int32),
            pltpu.VMEM((3, _QPW), jnp.float32),
            pltpu.SemaphoreType.DMA,
            pltpu.SemaphoreType.DMA,
        ],
    )
    out = run(points, new_points, featsT)
    return out.reshape(_B, _CH, _M, _S)
